# baseline (speedup 1.0000x reference)
import functools

import jax
import jax.numpy as jnp
from jax import lax
from jax.experimental import pallas as pl
from jax.experimental.pallas import tpu as pltpu
from jax.experimental.pallas import tpu_sc as plsc

B = 16384
F = 26
D = 16
M0 = 256
M1 = 128
EPS = 1e-5
V = 2600000

NC = 2
NS = 16
NW = NC * NS
L = 16

TOTAL_IDX = B * F
IDX_PER_W = TOTAL_IDX // NW
SUB = 128
NSUB = IDX_PER_W // SUB
GRP = 8
NGRP = NSUB // GRP
GIDX = GRP * SUB

VCHUNK = 1024
NCHUNK = V // VCHUNK
VREM = V - NCHUNK * VCHUNK
TPW = (NCHUNK + NW - 1) // NW
LINROWS = V * D // 128


def _mesh():
  return plsc.VectorSubcoreMesh(core_axis_name="c", subcore_axis_name="s",
                                num_cores=NC, num_subcores=NS)


LROWS_C = VCHUNK * D // 128


def _transpose_chunk(ib, ob, nrows):
  lane = lax.iota(jnp.int32, L)
  us = [jnp.full((L,), u, jnp.int32) for u in range(8)]

  @plsc.parallel_loop(0, nrows // 8, unroll=8 if nrows >= 64 else 1)
  def tr(v8):
    base = v8 * 8
    vals = [plsc.load_gather(ib, [lane, us[u] + base]) for u in range(8)]
    for u in range(8):
      ob[v8, pl.ds(u * D, D)] = vals[u]


def _detile_body(wt, lin, ib0, ib1, ob0, ob1, rb, ri,
                 sem_i0, sem_i1, sem_o0, sem_o1):
  wid = lax.axis_index("s") * NC + lax.axis_index("c")
  ibs = (ib0, ib1)
  obs = (ob0, ob1)
  isems = (sem_i0, sem_i1)
  osems = (sem_o0, sem_o1)

  @pl.when(wid < NCHUNK)
  def _():
    pltpu.async_copy(wt.at[:, pl.ds(wid * VCHUNK, VCHUNK)],
                     ib0.at[:, pl.ds(0, VCHUNK)], sem_i0)

  def step(t, carry):
    c = wid + t * NW

    for b in range(2):

      @pl.when((t % 2 == b) & (c < NCHUNK))
      def _():
        @pl.when(c + NW < NCHUNK)
        def _():
          pltpu.async_copy(wt.at[:, pl.ds((c + NW) * VCHUNK, VCHUNK)],
                           ibs[1 - b].at[:, pl.ds(0, VCHUNK)], isems[1 - b])

        pltpu.make_async_copy(
            wt.at[:, pl.ds(0, VCHUNK)], ibs[b].at[:, pl.ds(0, VCHUNK)],
            isems[b]).wait()

        @pl.when(t >= 2)
        def _():
          pltpu.make_async_copy(obs[b], lin.at[pl.ds(0, LROWS_C)],
                                osems[b]).wait()

        _transpose_chunk(ibs[b], obs[b], VCHUNK)
        pltpu.async_copy(obs[b], lin.at[pl.ds(c * LROWS_C, LROWS_C)],
                         osems[b])

    return carry

  lax.fori_loop(0, TPW, step, 0, unroll=False)

  nw_chunks = (NCHUNK - 1 - wid) // NW + 1

  for b in range(2):

    @pl.when(nw_chunks >= 1 + b)
    def _():
      pltpu.make_async_copy(obs[b], lin.at[pl.ds(0, LROWS_C)],
                            osems[b]).wait()

  @pl.when(wid == 0)
  def _():
    pltpu.sync_copy(wt.at[:, pl.ds(NCHUNK * VCHUNK, VREM)], ri)
    _transpose_chunk(ri, rb, VREM)
    pltpu.sync_copy(rb, lin.at[pl.ds(NCHUNK * LROWS_C, VREM * D // 128)])


@jax.jit
def _sc_detile(wt):
  return pl.kernel(
      _detile_body,
      out_type=jax.ShapeDtypeStruct((LINROWS, 128), jnp.float32),
      mesh=_mesh(),
      scratch_types=[
          pltpu.VMEM((D, VCHUNK + 1), jnp.float32),
          pltpu.VMEM((D, VCHUNK + 1), jnp.float32),
          pltpu.VMEM((LROWS_C, 128), jnp.float32),
          pltpu.VMEM((LROWS_C, 128), jnp.float32),
          pltpu.VMEM((VREM * D // 128, 128), jnp.float32),
          pltpu.VMEM((D, VREM), jnp.float32),
          pltpu.SemaphoreType.DMA,
          pltpu.SemaphoreType.DMA,
          pltpu.SemaphoreType.DMA,
          pltpu.SemaphoreType.DMA,
      ],
      compiler_params=pltpu.CompilerParams(use_tc_tiling_on_sc=True,
                                           needs_layout_passes=False),
  )(wt)


CB = 2048
NCB = (V + CB - 1) // CB


def _tc_detile_body(wt_ref, out_ref):
  x = wt_ref[...]
  y = x.reshape(D, CB // 8, 8).transpose(1, 2, 0)
  out_ref[...] = y.reshape(CB // 8, 128)


@jax.jit
def _tc_detile(wt):
  return pl.pallas_call(
      _tc_detile_body,
      grid=(NCB,),
      in_specs=[pl.BlockSpec((D, CB), lambda i: (0, i))],
      out_specs=pl.BlockSpec((CB // 8, 128), lambda i: (i, 0)),
      out_shape=jax.ShapeDtypeStruct((NCB * CB * D // 128, 128), jnp.float32),
  )(wt)


def _gath(ref, p0, cols):
  lane = lax.iota(jnp.int32, L)
  return plsc.load_gather(ref, [p0 + lane, cols])


def _sc_gather_body(x2d, wemb, wfc16, emb_out, fc_out,
                    idx_v, idxs_v, rows_v, fcraw_v, fcv_v, sem_e, sem_f):
  wid = lax.axis_index("s") * NC + lax.axis_index("c")
  base_sub = wid * NSUB
  base_idx = wid * IDX_PER_W
  pltpu.sync_copy(x2d.at[pl.ds(base_sub, NSUB)], idx_v)

  def shift_body(r, c):
    for k in range(SUB // L):
      v = idx_v[r, pl.ds(k * L, L)]
      idxs_v[r, pl.ds(k * L, L)] = lax.shift_right_logical(v, 4)
    return c

  lax.fori_loop(0, NSUB, shift_body, 0, unroll=False)

  def group(g, carry):
    descs = []
    for j in range(GRP):
      row = g * GRP + j
      ie = idx_v.at[row]
      iff = idxs_v.at[row]
      descs.append(pltpu.async_copy(
          wemb.at[ie], rows_v.at[pl.ds(j * SUB, SUB)], sem_e))
      descs.append(pltpu.async_copy(
          wfc16.at[iff], fcraw_v.at[pl.ds(j * SUB, SUB)], sem_f))
    for d in descs:
      d.wait()
    for j in range(GRP):
      row = g * GRP + j
      for k in range(SUB // L):
        p0 = j * SUB + k * L
        v = idx_v[row, pl.ds(k * L, L)]
        fcv_v[pl.ds(p0, L)] = _gath(fcraw_v, p0, lax.bitwise_and(v, 15))
    off = base_idx + g * GIDX
    pltpu.sync_copy(rows_v, emb_out.at[pl.ds(off, GIDX)])
    pltpu.sync_copy(fcv_v, fc_out.at[pl.ds(off, GIDX)])
    return carry

  lax.fori_loop(0, NGRP, group, 0, unroll=False)


def _sc_scratch():
  return [
      pltpu.VMEM((NSUB, SUB), jnp.int32),
      pltpu.VMEM((NSUB, SUB), jnp.int32),
      pltpu.VMEM((GIDX, D), jnp.float32),
      pltpu.VMEM((GIDX, D), jnp.float32),
      pltpu.VMEM((GIDX,), jnp.float32),
      pltpu.SemaphoreType.DMA,
      pltpu.SemaphoreType.DMA,
  ]


@jax.jit
def _sc_gather(x2d, wemb, wfc16):
  return pl.kernel(
      _sc_gather_body,
      out_type=(
          jax.ShapeDtypeStruct((TOTAL_IDX, D), jnp.float32),
          jax.ShapeDtypeStruct((TOTAL_IDX,), jnp.float32),
      ),
      mesh=_mesh(),
      scratch_types=_sc_scratch(),
      compiler_params=pltpu.CompilerParams(use_tc_tiling_on_sc=False,
                                           needs_layout_passes=False),
  )(x2d, wemb, wfc16)


BM = 1024


def _tc_body(conc_ref, fcm_ref, w0t_ref, b0_ref, w1t_ref, b1_ref, wl_ref,
             amat_ref, lw_ref, bt_ref, out_ref):
  c = conc_ref[...]
  h = jnp.dot(c, w0t_ref[...], preferred_element_type=jnp.float32)
  h = jnp.maximum(h + b0_ref[...], 0.0)
  h = jnp.dot(h, w1t_ref[...], preferred_element_type=jnp.float32)
  h = jnp.maximum(h + b1_ref[...], 0.0)
  mlp = jnp.sum(h * wl_ref[...], axis=1, keepdims=True)
  a = amat_ref[...]
  s = jnp.dot(c, a, preferred_element_type=jnp.float32)
  ss = jnp.dot(c * c, a, preferred_element_type=jnp.float32)
  fm = 0.5 * jnp.sum(s * s - ss, axis=1, keepdims=True)
  lin = jnp.sum(fcm_ref[...], axis=1, keepdims=True) * lw_ref[0, 0]
  out_ref[...] = jax.nn.sigmoid(lin + fm + mlp + bt_ref[0, 0])


def _tc_head(conc, fcm, w0t, b0f, w1t, b1f, wlr, amat, lw, btot):
  grid = (B // BM,)
  return pl.pallas_call(
      _tc_body,
      grid=grid,
      in_specs=[
          pl.BlockSpec((BM, F * D), lambda i: (i, 0)),
          pl.BlockSpec((BM, F), lambda i: (i, 0)),
          pl.BlockSpec((F * D, M0), lambda i: (0, 0)),
          pl.BlockSpec((1, M0), lambda i: (0, 0)),
          pl.BlockSpec((M0, M1), lambda i: (0, 0)),
          pl.BlockSpec((1, M1), lambda i: (0, 0)),
          pl.BlockSpec((1, M1), lambda i: (0, 0)),
          pl.BlockSpec((F * D, D), lambda i: (0, 0)),
          pl.BlockSpec(memory_space=pltpu.SMEM),
          pl.BlockSpec(memory_space=pltpu.SMEM),
      ],
      out_specs=pl.BlockSpec((BM, 1), lambda i: (i, 0)),
      out_shape=jax.ShapeDtypeStruct((B, 1), jnp.float32),
  )(conc, fcm, w0t, b0f, w1t, b1f, wlr, amat, lw, btot)


def kernel(x, W_emb, W_fc, lin_w, lin_b, W0, b0, g0, bt0, rm0, rv0,
           W1, b1, g1, bt1, rm1, rv1, Wl, bl):
  x2d = x.reshape(TOTAL_IDX // SUB, SUB).astype(jnp.int32)
  lin_tbl = _tc_detile(W_emb.T)
  wembL = lin_tbl.reshape(-1, D)
  wfc16 = W_fc.reshape(-1, 16)
  emb_flat, fc_flat = _sc_gather(x2d, wembL, wfc16)
  conc = emb_flat.reshape(B, F * D)
  fcm = fc_flat.reshape(B, F)
  s0 = g0 * lax.rsqrt(rv0 + EPS)
  w0t = (W0 * s0[:, None]).T
  b0f = ((b0 - rm0) * s0 + bt0).reshape(1, M0)
  s1 = g1 * lax.rsqrt(rv1 + EPS)
  w1t = (W1 * s1[:, None]).T
  b1f = ((b1 - rm1) * s1 + bt1).reshape(1, M1)
  wlr = Wl.reshape(1, M1)
  amat = jnp.tile(jnp.eye(D, dtype=jnp.float32), (F, 1))
  btot = (lin_b + bl).reshape(1, 1)
  out = _tc_head(conc, fcm, w0t, b0f, w1t, b1f, wlr, amat, lin_w, btot)
  return out[:, 0]


# speedup vs baseline: 3.7732x; 3.7732x over previous
import functools

import jax
import jax.numpy as jnp
from jax import lax
from jax.experimental import pallas as pl
from jax.experimental.pallas import tpu as pltpu
from jax.experimental.pallas import tpu_sc as plsc

B = 16384
F = 26
D = 16
M0 = 256
M1 = 128
EPS = 1e-5
V = 2600000

NC = 2
NS = 16
NW = NC * NS
L = 16

TOTAL_IDX = B * F
IDX_PER_W = TOTAL_IDX // NW
SUB = 128
NSUB = IDX_PER_W // SUB
GRP = 8
NGRP = NSUB // GRP
GIDX = GRP * SUB

VCHUNK = 1024
NCHUNK = V // VCHUNK
VREM = V - NCHUNK * VCHUNK
TPW = (NCHUNK + NW - 1) // NW
LINROWS = V * D // 128


def _mesh():
  return plsc.VectorSubcoreMesh(core_axis_name="c", subcore_axis_name="s",
                                num_cores=NC, num_subcores=NS)


LROWS_C = VCHUNK * D // 128


def _transpose_chunk(ib, ob, nrows):
  lane = lax.iota(jnp.int32, L)
  us = [jnp.full((L,), u, jnp.int32) for u in range(8)]

  @plsc.parallel_loop(0, nrows // 8, unroll=8 if nrows >= 64 else 1)
  def tr(v8):
    base = v8 * 8
    vals = [plsc.load_gather(ib, [lane, us[u] + base]) for u in range(8)]
    for u in range(8):
      ob[v8, pl.ds(u * D, D)] = vals[u]


def _detile_body(wt, lin, ib0, ib1, ob0, ob1, rb, ri,
                 sem_i0, sem_i1, sem_o0, sem_o1):
  wid = lax.axis_index("s") * NC + lax.axis_index("c")
  ibs = (ib0, ib1)
  obs = (ob0, ob1)
  isems = (sem_i0, sem_i1)
  osems = (sem_o0, sem_o1)

  @pl.when(wid < NCHUNK)
  def _():
    pltpu.async_copy(wt.at[:, pl.ds(wid * VCHUNK, VCHUNK)],
                     ib0.at[:, pl.ds(0, VCHUNK)], sem_i0)

  def step(t, carry):
    c = wid + t * NW

    for b in range(2):

      @pl.when((t % 2 == b) & (c < NCHUNK))
      def _():
        @pl.when(c + NW < NCHUNK)
        def _():
          pltpu.async_copy(wt.at[:, pl.ds((c + NW) * VCHUNK, VCHUNK)],
                           ibs[1 - b].at[:, pl.ds(0, VCHUNK)], isems[1 - b])

        pltpu.make_async_copy(
            wt.at[:, pl.ds(0, VCHUNK)], ibs[b].at[:, pl.ds(0, VCHUNK)],
            isems[b]).wait()

        @pl.when(t >= 2)
        def _():
          pltpu.make_async_copy(obs[b], lin.at[pl.ds(0, LROWS_C)],
                                osems[b]).wait()

        _transpose_chunk(ibs[b], obs[b], VCHUNK)
        pltpu.async_copy(obs[b], lin.at[pl.ds(c * LROWS_C, LROWS_C)],
                         osems[b])

    return carry

  lax.fori_loop(0, TPW, step, 0, unroll=False)

  nw_chunks = (NCHUNK - 1 - wid) // NW + 1

  for b in range(2):

    @pl.when(nw_chunks >= 1 + b)
    def _():
      pltpu.make_async_copy(obs[b], lin.at[pl.ds(0, LROWS_C)],
                            osems[b]).wait()

  @pl.when(wid == 0)
  def _():
    pltpu.sync_copy(wt.at[:, pl.ds(NCHUNK * VCHUNK, VREM)], ri)
    _transpose_chunk(ri, rb, VREM)
    pltpu.sync_copy(rb, lin.at[pl.ds(NCHUNK * LROWS_C, VREM * D // 128)])


@jax.jit
def _sc_detile(wt):
  return pl.kernel(
      _detile_body,
      out_type=jax.ShapeDtypeStruct((LINROWS, 128), jnp.float32),
      mesh=_mesh(),
      scratch_types=[
          pltpu.VMEM((D, VCHUNK + 1), jnp.float32),
          pltpu.VMEM((D, VCHUNK + 1), jnp.float32),
          pltpu.VMEM((LROWS_C, 128), jnp.float32),
          pltpu.VMEM((LROWS_C, 128), jnp.float32),
          pltpu.VMEM((VREM * D // 128, 128), jnp.float32),
          pltpu.VMEM((D, VREM), jnp.float32),
          pltpu.SemaphoreType.DMA,
          pltpu.SemaphoreType.DMA,
          pltpu.SemaphoreType.DMA,
          pltpu.SemaphoreType.DMA,
      ],
      compiler_params=pltpu.CompilerParams(use_tc_tiling_on_sc=True,
                                           needs_layout_passes=False),
  )(wt)


def _gath(ref, p0, cols):
  lane = lax.iota(jnp.int32, L)
  return plsc.load_gather(ref, [p0 + lane, cols])


def _sc_gather_body(x2d, wemb, wfc16, emb_out, fc_out,
                    idx_v, idxs_v, rows_v, fcraw_v, fcv_v, sem_e, sem_f):
  wid = lax.axis_index("s") * NC + lax.axis_index("c")
  base_sub = wid * NSUB
  base_idx = wid * IDX_PER_W
  pltpu.sync_copy(x2d.at[pl.ds(base_sub, NSUB)], idx_v)

  def shift_body(r, c):
    for k in range(SUB // L):
      v = idx_v[r, pl.ds(k * L, L)]
      idxs_v[r, pl.ds(k * L, L)] = lax.shift_right_logical(v, 4)
    return c

  lax.fori_loop(0, NSUB, shift_body, 0, unroll=False)

  def group(g, carry):
    descs = []
    for j in range(GRP):
      row = g * GRP + j
      ie = idx_v.at[row]
      iff = idxs_v.at[row]
      descs.append(pltpu.async_copy(
          wemb.at[ie], rows_v.at[pl.ds(j * SUB, SUB)], sem_e))
      descs.append(pltpu.async_copy(
          wfc16.at[iff], fcraw_v.at[pl.ds(j * SUB, SUB)], sem_f))
    for d in descs:
      d.wait()
    for j in range(GRP):
      row = g * GRP + j
      for k in range(SUB // L):
        p0 = j * SUB + k * L
        v = idx_v[row, pl.ds(k * L, L)]
        fcv_v[pl.ds(p0, L)] = _gath(fcraw_v, p0, lax.bitwise_and(v, 15))
    off = base_idx + g * GIDX
    pltpu.sync_copy(rows_v, emb_out.at[pl.ds(off, GIDX)])
    pltpu.sync_copy(fcv_v, fc_out.at[pl.ds(off, GIDX)])
    return carry

  lax.fori_loop(0, NGRP, group, 0, unroll=False)


def _sc_scratch():
  return [
      pltpu.VMEM((NSUB, SUB), jnp.int32),
      pltpu.VMEM((NSUB, SUB), jnp.int32),
      pltpu.VMEM((GIDX, D), jnp.float32),
      pltpu.VMEM((GIDX, D), jnp.float32),
      pltpu.VMEM((GIDX,), jnp.float32),
      pltpu.SemaphoreType.DMA,
      pltpu.SemaphoreType.DMA,
  ]


@jax.jit
def _sc_gather(x2d, wemb, wfc16):
  return pl.kernel(
      _sc_gather_body,
      out_type=(
          jax.ShapeDtypeStruct((TOTAL_IDX, D), jnp.float32),
          jax.ShapeDtypeStruct((TOTAL_IDX,), jnp.float32),
      ),
      mesh=_mesh(),
      scratch_types=_sc_scratch(),
      compiler_params=pltpu.CompilerParams(use_tc_tiling_on_sc=False,
                                           needs_layout_passes=False),
  )(x2d, wemb, wfc16)


BM = 1024


def _tc_body(conc_ref, fcm_ref, w0t_ref, b0_ref, w1t_ref, b1_ref, wl_ref,
             amat_ref, lw_ref, bt_ref, out_ref):
  c = conc_ref[...]
  h = jnp.dot(c, w0t_ref[...], preferred_element_type=jnp.float32)
  h = jnp.maximum(h + b0_ref[...], 0.0)
  h = jnp.dot(h, w1t_ref[...], preferred_element_type=jnp.float32)
  h = jnp.maximum(h + b1_ref[...], 0.0)
  mlp = jnp.sum(h * wl_ref[...], axis=1, keepdims=True)
  a = amat_ref[...]
  s = jnp.dot(c, a, preferred_element_type=jnp.float32)
  ss = jnp.dot(c * c, a, preferred_element_type=jnp.float32)
  fm = 0.5 * jnp.sum(s * s - ss, axis=1, keepdims=True)
  lin = jnp.sum(fcm_ref[...], axis=1, keepdims=True) * lw_ref[0, 0]
  out_ref[...] = jax.nn.sigmoid(lin + fm + mlp + bt_ref[0, 0])


def _tc_head(conc, fcm, w0t, b0f, w1t, b1f, wlr, amat, lw, btot):
  grid = (B // BM,)
  return pl.pallas_call(
      _tc_body,
      grid=grid,
      in_specs=[
          pl.BlockSpec((BM, F * D), lambda i: (i, 0)),
          pl.BlockSpec((BM, F), lambda i: (i, 0)),
          pl.BlockSpec((F * D, M0), lambda i: (0, 0)),
          pl.BlockSpec((1, M0), lambda i: (0, 0)),
          pl.BlockSpec((M0, M1), lambda i: (0, 0)),
          pl.BlockSpec((1, M1), lambda i: (0, 0)),
          pl.BlockSpec((1, M1), lambda i: (0, 0)),
          pl.BlockSpec((F * D, D), lambda i: (0, 0)),
          pl.BlockSpec(memory_space=pltpu.SMEM),
          pl.BlockSpec(memory_space=pltpu.SMEM),
      ],
      out_specs=pl.BlockSpec((BM, 1), lambda i: (i, 0)),
      out_shape=jax.ShapeDtypeStruct((B, 1), jnp.float32),
  )(conc, fcm, w0t, b0f, w1t, b1f, wlr, amat, lw, btot)


def kernel(x, W_emb, W_fc, lin_w, lin_b, W0, b0, g0, bt0, rm0, rv0,
           W1, b1, g1, bt1, rm1, rv1, Wl, bl):
  x2d = x.reshape(TOTAL_IDX // SUB, SUB).astype(jnp.int32)
  lin_tbl = _sc_detile(W_emb.T)
  wembL = lin_tbl.reshape(V, D)
  wfc16 = W_fc.reshape(-1, 16)
  emb_flat, fc_flat = _sc_gather(x2d, wembL, wfc16)
  conc = emb_flat.reshape(B, F * D)
  fcm = fc_flat.reshape(B, F)
  s0 = g0 * lax.rsqrt(rv0 + EPS)
  w0t = (W0 * s0[:, None]).T
  b0f = ((b0 - rm0) * s0 + bt0).reshape(1, M0)
  s1 = g1 * lax.rsqrt(rv1 + EPS)
  w1t = (W1 * s1[:, None]).T
  b1f = ((b1 - rm1) * s1 + bt1).reshape(1, M1)
  wlr = Wl.reshape(1, M1)
  amat = jnp.tile(jnp.eye(D, dtype=jnp.float32), (F, 1))
  btot = (lin_b + bl).reshape(1, 1)
  out = _tc_head(conc, fcm, w0t, b0f, w1t, b1f, wlr, amat, lin_w, btot)
  return out[:, 0]
